# baseline (speedup 1.0000x reference)
import functools

import jax
import jax.numpy as jnp
from jax import lax
from jax.experimental import pallas as pl
from jax.experimental.pallas import tpu as pltpu
from jax.experimental.pallas import tpu_sc as plsc

N = 10000
E = 320000
NFEAT = 128
NHID = 16
NHEADS = 8
HD = NHEADS * NHID
TW = HD + 16
ALPHA = 0.2
EPS = 1e-16

NW = 32
CHUNK = 64
NCHUNKS = E // CHUNK
NRCH = N // CHUNK
NRTAIL = N - NRCH * CHUNK


def _tc_proj_body(x_ref, wc_ref, a1_ref, a2_ref, h_ref, s1_ref, s2_ref):
    h = jnp.dot(x_ref[...], wc_ref[...], preferred_element_type=jnp.float32)
    h_ref[...] = h
    s1_ref[...] = jnp.dot(h, a1_ref[...], preferred_element_type=jnp.float32)
    s2_ref[...] = jnp.dot(h, a2_ref[...], preferred_element_type=jnp.float32)


def _tc_proj(x, wc, a1, a2):
    return pl.pallas_call(
        _tc_proj_body,
        out_shape=(
            jax.ShapeDtypeStruct((N, HD), jnp.float32),
            jax.ShapeDtypeStruct((N, NHID), jnp.float32),
            jax.ShapeDtypeStruct((N, NHID), jnp.float32),
        ),
    )(x, wc, a1, a2)


def _tc_mid_body(ph_ref, pw_ref, srep_ref, wout_ref, a1_ref, a2_ref,
                 h2_ref, s1_ref, s2_ref):
    hsum = ph_ref[0] + ph_ref[1]
    wsum = pw_ref[0] + pw_ref[1]
    wfull = jnp.dot(wsum, srep_ref[...], preferred_element_type=jnp.float32)
    hp = hsum / (wfull + EPS)
    x1 = jnp.where(hp > 0, hp, jnp.exp(hp) - 1.0)
    h2 = jnp.dot(x1, wout_ref[...], preferred_element_type=jnp.float32)
    h2_ref[...] = h2
    s1_ref[...] = jnp.dot(h2, a1_ref[...], preferred_element_type=jnp.float32)
    s2_ref[...] = jnp.dot(h2, a2_ref[...], preferred_element_type=jnp.float32)


def _tc_mid(ph, pw, srep, wout, a1, a2):
    return pl.pallas_call(
        _tc_mid_body,
        out_shape=(
            jax.ShapeDtypeStruct((N, HD), jnp.float32),
            jax.ShapeDtypeStruct((N, NHID), jnp.float32),
            jax.ShapeDtypeStruct((N, NHID), jnp.float32),
        ),
    )(ph, pw, srep, wout, a1, a2)


def _tc_out_body(ph_ref, pw_ref, s0_ref, out_ref):
    hsum = ph_ref[0] + ph_ref[1]
    wsum = pw_ref[0] + pw_ref[1]
    wfull = jnp.dot(wsum, s0_ref[...], preferred_element_type=jnp.float32)
    hp = hsum / (wfull + EPS)
    out_ref[...] = jnp.where(hp > 0, hp, jnp.exp(hp) - 1.0)


def _tc_out(ph, pw, s0):
    return pl.pallas_call(
        _tc_out_body,
        out_shape=jax.ShapeDtypeStruct((N, HD), jnp.float32),
    )(ph, pw, s0)


_SPLAT_DNUMS = lax.GatherDimensionNumbers(
    offset_dims=(), collapsed_slice_dims=(0,), start_index_map=(0,))

NBUF = 2
NIBUF = 4
_SLOTS_OUTER = (NCHUNKS // NW + 1 + 2 + 3) // 4 + 1


def _make_edge_pass(nheads):
    widx = [j if nheads == NHEADS else 0 for j in range(NHEADS)]
    mesh = plsc.VectorSubcoreMesh(core_axis_name="c", subcore_axis_name="s")

    per_buf = [
        pltpu.VMEM((CHUNK,), jnp.int32),
        pltpu.VMEM((CHUNK, TW), jnp.float32),
        pltpu.VMEM((CHUNK, HD), jnp.float32),
        pltpu.VMEM((CHUNK, NHID), jnp.float32),
        pltpu.VMEM((CHUNK, NHID), jnp.float32),
        pltpu.SemaphoreType.DMA,
        pltpu.SemaphoreType.DMA,
    ]
    npb = len(per_buf)
    per_ibuf = [
        pltpu.VMEM((2 * CHUNK,), jnp.int32),
        pltpu.SemaphoreType.DMA,
    ]
    nipb = len(per_ibuf)

    @functools.partial(
        pl.kernel,
        out_type=(
            jax.ShapeDtypeStruct((2, N, HD), jnp.float32),
            jax.ShapeDtypeStruct((2, N, NHID), jnp.float32),
        ),
        mesh=mesh,
        scratch_types=per_buf * NBUF + per_ibuf * NIBUF + [
            pltpu.VMEM_SHARED((N, HD), jnp.float32),
            pltpu.VMEM_SHARED((N, NHID), jnp.float32),
        ],
        compiler_params=pltpu.CompilerParams(use_tc_tiling_on_sc=False),
    )
    def edge_pass(packed_hbm, h_hbm, s1_hbm, outh_hbm, outw_hbm,
                  *scratch):
        bufs = [scratch[i * npb:(i + 1) * npb] for i in range(NBUF)]
        ioff = npb * NBUF
        ibufs = [scratch[ioff + i * nipb:ioff + (i + 1) * nipb]
                 for i in range(NIBUF)]
        acc_h, acc_w = scratch[ioff + nipb * NIBUF], scratch[ioff + nipb * NIBUF + 1]
        cid = lax.axis_index("c")
        sid = lax.axis_index("s")
        wid = sid * 2 + cid

        lane = lax.iota(jnp.int32, 16)
        headmask = lane < nheads
        zero16 = jnp.zeros((16,), jnp.float32)

        zh, zw = bufs[0][2], bufs[0][4]

        def zbody(i, _):
            for j in range(NHEADS):
                zh[i, pl.ds(j * 16, 16)] = zero16
            zw[i] = zero16
            return 0
        lax.fori_loop(0, CHUNK, zbody, 0)
        for t in range(NRCH // 16 + 1):
            rc = sid + 16 * t

            @pl.when(rc < NRCH)
            def _():
                pltpu.sync_copy(zh, acc_h.at[pl.ds(rc * CHUNK, CHUNK)])
                pltpu.sync_copy(zw, acc_w.at[pl.ds(rc * CHUNK, CHUNK)])

        @pl.when(sid == NRCH % 16)
        def _():
            pltpu.sync_copy(zh.at[pl.ds(0, NRTAIL)],
                            acc_h.at[pl.ds(NRCH * CHUNK, NRTAIL)])
            pltpu.sync_copy(zw.at[pl.ds(0, NRTAIL)],
                            acc_w.at[pl.ds(NRCH * CHUNK, NRTAIL)])
        plsc.subcore_barrier()

        nch = jnp.where(wid < NCHUNKS - (NCHUNKS // NW) * NW,
                        NCHUNKS // NW + 1, NCHUNKS // NW)

        def issue_idx(i4, k):
            idx2, semi = ibufs[i4]
            pltpu.async_copy(packed_hbm.at[k * NW + wid], idx2, semi)

        def wait_idx(i4, k):
            idx2, semi = ibufs[i4]
            pltpu.make_async_copy(packed_hbm.at[k * NW + wid], idx2, semi).wait()

        def issue_gathers(b, i4):
            _, hbuf, _, s1v, _, semg, _ = bufs[b]
            idx2 = ibufs[i4][0]
            pltpu.async_copy(h_hbm.at[idx2.at[pl.ds(CHUNK, CHUNK)]], hbuf, semg)
            pltpu.async_copy(s1_hbm.at[idx2.at[pl.ds(0, CHUNK)]], s1v, semg)

        def wait_gathers(b, i4):
            _, hbuf, _, s1v, _, semg, _ = bufs[b]
            idx2 = ibufs[i4][0]
            pltpu.make_async_copy(
                h_hbm.at[idx2.at[pl.ds(CHUNK, CHUNK)]], hbuf, semg).wait()
            pltpu.make_async_copy(
                s1_hbm.at[idx2.at[pl.ds(0, CHUNK)]], s1v, semg).wait()

        def compute(b, i4):
            sidx, hbuf, obuf, s1v, wrow, _, _ = bufs[b]
            idx2 = ibufs[i4][0]
            for t in range(CHUNK // 16):
                sidx[pl.ds(t * 16, 16)] = idx2[pl.ds(t * 16, 16)]

            @plsc.parallel_loop(0, CHUNK, unroll=4)
            def _(e):
                c = s1v[e] + hbuf[e, pl.ds(HD, 16)]
                cl = jnp.where(c > 0, c, ALPHA * c)
                w = jnp.exp(-cl)
                w = jnp.where(headmask, w, 0.0)
                wrow[e] = w
                if nheads == 1:
                    wj = lax.gather(
                        w, jnp.zeros((16, 1), jnp.int32),
                        _SPLAT_DNUMS, (1,),
                        mode=lax.GatherScatterMode.PROMISE_IN_BOUNDS)
                    for j in range(NHEADS):
                        obuf[e, pl.ds(j * 16, 16)] = (
                            wj * hbuf[e, pl.ds(j * 16, 16)])
                else:
                    for j in range(NHEADS):
                        wj = lax.gather(
                            w, jnp.full((16, 1), widx[j], jnp.int32),
                            _SPLAT_DNUMS, (1,),
                            mode=lax.GatherScatterMode.PROMISE_IN_BOUNDS)
                        obuf[e, pl.ds(j * 16, 16)] = (
                            wj * hbuf[e, pl.ds(j * 16, 16)])

        def issue_scatters(b):
            sidx, _, obuf, _, wrow, _, sems = bufs[b]
            pltpu.async_copy(obuf, acc_h.at[sidx], sems, add=True)
            pltpu.async_copy(wrow, acc_w.at[sidx], sems, add=True)

        def wait_scatters(b):
            sidx, _, obuf, _, wrow, _, sems = bufs[b]
            pltpu.make_async_copy(obuf, acc_h.at[sidx], sems).wait()
            pltpu.make_async_copy(wrow, acc_w.at[sidx], sems).wait()

        for k0 in range(NBUF):
            pltpu.sync_copy(packed_hbm.at[k0 * NW + wid], ibufs[k0][0])
            issue_gathers(k0 % NBUF, k0)
        for k0 in range(NBUF, NIBUF):
            issue_idx(k0, k0)

        def slot_body(g, _):
            for u in range(4):
                k = g * 4 + u
                b = u % NBUF
                i4 = u

                @pl.when(k < nch)
                def _():
                    wait_gathers(b, i4)

                @pl.when(jnp.logical_and(k >= NBUF, k - NBUF < nch))
                def _():
                    wait_scatters(b)

                @pl.when(k < nch)
                def _():
                    compute(b, i4)
                    issue_scatters(b)

                @pl.when(k + NIBUF < nch)
                def _():
                    issue_idx(i4, k + NIBUF)

                @pl.when(k + NBUF < nch)
                def _():
                    wait_idx((u + NBUF) % NIBUF, k + NBUF)
                    issue_gathers(b, (u + NBUF) % NIBUF)
            return 0
        lax.fori_loop(0, _SLOTS_OUTER, slot_body, 0)
        plsc.subcore_barrier()

        for t in range(NRCH // 16 + 1):
            rc = sid + 16 * t

            @pl.when(rc < NRCH)
            def _():
                pltpu.sync_copy(acc_h.at[pl.ds(rc * CHUNK, CHUNK)],
                                outh_hbm.at[cid, pl.ds(rc * CHUNK, CHUNK)])
                pltpu.sync_copy(acc_w.at[pl.ds(rc * CHUNK, CHUNK)],
                                outw_hbm.at[cid, pl.ds(rc * CHUNK, CHUNK)])

        @pl.when(sid == NRCH % 16)
        def _():
            pltpu.sync_copy(acc_h.at[pl.ds(NRCH * CHUNK, NRTAIL)],
                            outh_hbm.at[cid, pl.ds(NRCH * CHUNK, NRTAIL)])
            pltpu.sync_copy(acc_w.at[pl.ds(NRCH * CHUNK, NRTAIL)],
                            outw_hbm.at[cid, pl.ds(NRCH * CHUNK, NRTAIL)])

    return edge_pass


_edge_pass8 = _make_edge_pass(NHEADS)
_edge_pass1 = _make_edge_pass(1)


def kernel(Corpus_, batch_inputs, entity_embeddings, edge_list, W, a, W_out, a_out):
    src = edge_list[0].astype(jnp.int32)
    dst = edge_list[1].astype(jnp.int32)
    x = entity_embeddings

    wcat = jnp.transpose(W, (1, 0, 2)).reshape(NFEAT, HD)
    aL = a[:, 0, :NHID]
    aR = a[:, 0, NHID:]
    eyeh = jnp.eye(NHEADS, dtype=jnp.float32)
    a1 = jnp.concatenate(
        [(aL[:, :, None] * eyeh[:, None, :]).reshape(HD, NHEADS),
         jnp.zeros((HD, NHEADS), jnp.float32)], axis=1)
    a2 = jnp.concatenate(
        [(aR[:, :, None] * eyeh[:, None, :]).reshape(HD, NHEADS),
         jnp.zeros((HD, NHEADS), jnp.float32)], axis=1)
    srep = jnp.concatenate(
        [jnp.repeat(eyeh, NHID, axis=1), jnp.zeros((NHEADS, HD), jnp.float32)],
        axis=0)
    a1o = jnp.concatenate(
        [a_out[0, :HD][:, None], jnp.zeros((HD, NHID - 1), jnp.float32)], axis=1)
    a2o = jnp.concatenate(
        [a_out[0, HD:][:, None], jnp.zeros((HD, NHID - 1), jnp.float32)], axis=1)
    s0 = jnp.concatenate(
        [jnp.ones((1, HD), jnp.float32), jnp.zeros((NHID - 1, HD), jnp.float32)],
        axis=0)

    packed = jnp.concatenate(
        [src.reshape(NCHUNKS, CHUNK), dst.reshape(NCHUNKS, CHUNK)], axis=1)

    h1, s1t, s2t = _tc_proj(x, wcat, a1, a2)
    ph1, pw1 = _edge_pass8(packed, jnp.concatenate([h1, s2t], axis=1), s1t)
    h2, s1t2, s2t2 = _tc_mid(ph1, pw1, srep, W_out, a1o, a2o)
    ph2, pw2 = _edge_pass1(packed, jnp.concatenate([h2, s2t2], axis=1), s1t2)
    return _tc_out(ph2, pw2, s0)


# speedup vs baseline: 1.0629x; 1.0629x over previous
import functools

import jax
import jax.numpy as jnp
from jax import lax
from jax.experimental import pallas as pl
from jax.experimental.pallas import tpu as pltpu
from jax.experimental.pallas import tpu_sc as plsc

N = 10000
E = 320000
NFEAT = 128
NHID = 16
NHEADS = 8
HD = NHEADS * NHID
ALPHA = 0.2
EPS = 1e-16

NW = 32
CHUNK = 64
NCHUNKS = E // CHUNK
NRCH = N // CHUNK
NRTAIL = N - NRCH * CHUNK


def _tc_proj_body(x_ref, wc_ref, a1_ref, a2_ref, h_ref, s1_ref, s2_ref):
    h = jnp.dot(x_ref[...], wc_ref[...], preferred_element_type=jnp.float32)
    h_ref[...] = h
    s1_ref[...] = jnp.dot(h, a1_ref[...], preferred_element_type=jnp.float32)
    s2_ref[...] = jnp.dot(h, a2_ref[...], preferred_element_type=jnp.float32)


def _tc_proj(x, wc, a1, a2):
    return pl.pallas_call(
        _tc_proj_body,
        out_shape=(
            jax.ShapeDtypeStruct((N, HD), jnp.float32),
            jax.ShapeDtypeStruct((N, NHID), jnp.float32),
            jax.ShapeDtypeStruct((N, NHID), jnp.float32),
        ),
    )(x, wc, a1, a2)


def _tc_mid_body(ph_ref, pw_ref, srep_ref, wout_ref, a1_ref, a2_ref,
                 h2_ref, s1_ref, s2_ref):
    hsum = ph_ref[0] + ph_ref[1]
    wsum = pw_ref[0] + pw_ref[1]
    wfull = jnp.dot(wsum, srep_ref[...], preferred_element_type=jnp.float32)
    hp = hsum / (wfull + EPS)
    x1 = jnp.where(hp > 0, hp, jnp.exp(hp) - 1.0)
    h2 = jnp.dot(x1, wout_ref[...], preferred_element_type=jnp.float32)
    h2_ref[...] = h2
    s1_ref[...] = jnp.dot(h2, a1_ref[...], preferred_element_type=jnp.float32)
    s2_ref[...] = jnp.dot(h2, a2_ref[...], preferred_element_type=jnp.float32)


def _tc_mid(ph, pw, srep, wout, a1, a2):
    return pl.pallas_call(
        _tc_mid_body,
        out_shape=(
            jax.ShapeDtypeStruct((N, HD), jnp.float32),
            jax.ShapeDtypeStruct((N, NHID), jnp.float32),
            jax.ShapeDtypeStruct((N, NHID), jnp.float32),
        ),
    )(ph, pw, srep, wout, a1, a2)


def _tc_out_body(ph_ref, pw_ref, s0_ref, out_ref):
    hsum = ph_ref[0] + ph_ref[1]
    wsum = pw_ref[0] + pw_ref[1]
    wfull = jnp.dot(wsum, s0_ref[...], preferred_element_type=jnp.float32)
    hp = hsum / (wfull + EPS)
    out_ref[...] = jnp.where(hp > 0, hp, jnp.exp(hp) - 1.0)


def _tc_out(ph, pw, s0):
    return pl.pallas_call(
        _tc_out_body,
        out_shape=jax.ShapeDtypeStruct((N, HD), jnp.float32),
    )(ph, pw, s0)


_SPLAT_DNUMS = lax.GatherDimensionNumbers(
    offset_dims=(), collapsed_slice_dims=(0,), start_index_map=(0,))

NBUF = 2
NIBUF = 4
_SLOTS_OUTER = (NCHUNKS // NW + 1 + 2 + 3) // 4 + 1


def _make_edge_pass(nheads):
    widx = [j if nheads == NHEADS else 0 for j in range(NHEADS)]
    mesh = plsc.VectorSubcoreMesh(core_axis_name="c", subcore_axis_name="s")

    per_buf = [
        pltpu.VMEM((CHUNK,), jnp.int32),
        pltpu.VMEM((CHUNK, HD), jnp.float32),
        pltpu.VMEM((CHUNK, HD), jnp.float32),
        pltpu.VMEM((CHUNK, NHID), jnp.float32),
        pltpu.VMEM((CHUNK, NHID), jnp.float32),
        pltpu.VMEM((CHUNK, NHID), jnp.float32),
        pltpu.SemaphoreType.DMA,
        pltpu.SemaphoreType.DMA,
    ]
    npb = len(per_buf)
    per_ibuf = [
        pltpu.VMEM((2 * CHUNK,), jnp.int32),
        pltpu.SemaphoreType.DMA,
    ]
    nipb = len(per_ibuf)

    @functools.partial(
        pl.kernel,
        out_type=(
            jax.ShapeDtypeStruct((2, N, HD), jnp.float32),
            jax.ShapeDtypeStruct((2, N, NHID), jnp.float32),
        ),
        mesh=mesh,
        scratch_types=per_buf * NBUF + per_ibuf * NIBUF + [
            pltpu.VMEM_SHARED((N, HD), jnp.float32),
            pltpu.VMEM_SHARED((N, NHID), jnp.float32),
        ],
        compiler_params=pltpu.CompilerParams(use_tc_tiling_on_sc=False),
    )
    def edge_pass(packed_hbm, h_hbm, s1_hbm, s2_hbm, outh_hbm, outw_hbm,
                  *scratch):
        bufs = [scratch[i * npb:(i + 1) * npb] for i in range(NBUF)]
        ioff = npb * NBUF
        ibufs = [scratch[ioff + i * nipb:ioff + (i + 1) * nipb]
                 for i in range(NIBUF)]
        acc_h, acc_w = scratch[ioff + nipb * NIBUF], scratch[ioff + nipb * NIBUF + 1]
        cid = lax.axis_index("c")
        sid = lax.axis_index("s")
        wid = sid * 2 + cid

        lane = lax.iota(jnp.int32, 16)
        headmask = lane < nheads
        zero16 = jnp.zeros((16,), jnp.float32)

        zh, zw = bufs[0][1], bufs[0][5]

        def zbody(i, _):
            for j in range(NHEADS):
                zh[i, pl.ds(j * 16, 16)] = zero16
            zw[i] = zero16
            return 0
        lax.fori_loop(0, CHUNK, zbody, 0)
        for t in range(NRCH // 16 + 1):
            rc = sid + 16 * t

            @pl.when(rc < NRCH)
            def _():
                pltpu.sync_copy(zh, acc_h.at[pl.ds(rc * CHUNK, CHUNK)])
                pltpu.sync_copy(zw, acc_w.at[pl.ds(rc * CHUNK, CHUNK)])

        @pl.when(sid == NRCH % 16)
        def _():
            pltpu.sync_copy(zh.at[pl.ds(0, NRTAIL)],
                            acc_h.at[pl.ds(NRCH * CHUNK, NRTAIL)])
            pltpu.sync_copy(zw.at[pl.ds(0, NRTAIL)],
                            acc_w.at[pl.ds(NRCH * CHUNK, NRTAIL)])
        plsc.subcore_barrier()

        nch = jnp.where(wid < NCHUNKS - (NCHUNKS // NW) * NW,
                        NCHUNKS // NW + 1, NCHUNKS // NW)

        def issue_idx(i4, k):
            idx2, semi = ibufs[i4]
            pltpu.async_copy(packed_hbm.at[k * NW + wid], idx2, semi)

        def wait_idx(i4, k):
            idx2, semi = ibufs[i4]
            pltpu.make_async_copy(packed_hbm.at[k * NW + wid], idx2, semi).wait()

        def issue_gathers(b, i4):
            _, hbuf, _, s1v, s2v, _, semg, _ = bufs[b]
            idx2 = ibufs[i4][0]
            half = CHUNK // 2
            pltpu.async_copy(h_hbm.at[idx2.at[pl.ds(CHUNK, half)]],
                             hbuf.at[pl.ds(0, half)], semg)
            pltpu.async_copy(h_hbm.at[idx2.at[pl.ds(CHUNK + half, half)]],
                             hbuf.at[pl.ds(half, half)], semg)
            pltpu.async_copy(s1_hbm.at[idx2.at[pl.ds(0, CHUNK)]], s1v, semg)
            pltpu.async_copy(s2_hbm.at[idx2.at[pl.ds(CHUNK, CHUNK)]], s2v, semg)

        def wait_gathers(b, i4):
            _, hbuf, _, s1v, s2v, _, semg, _ = bufs[b]
            idx2 = ibufs[i4][0]
            half = CHUNK // 2
            pltpu.make_async_copy(
                h_hbm.at[idx2.at[pl.ds(CHUNK, half)]],
                hbuf.at[pl.ds(0, half)], semg).wait()
            pltpu.make_async_copy(
                h_hbm.at[idx2.at[pl.ds(CHUNK + half, half)]],
                hbuf.at[pl.ds(half, half)], semg).wait()
            pltpu.make_async_copy(
                s1_hbm.at[idx2.at[pl.ds(0, CHUNK)]], s1v, semg).wait()
            pltpu.make_async_copy(
                s2_hbm.at[idx2.at[pl.ds(CHUNK, CHUNK)]], s2v, semg).wait()

        def compute(b, i4):
            sidx, hbuf, obuf, s1v, s2v, wrow, _, _ = bufs[b]
            idx2 = ibufs[i4][0]
            for t in range(CHUNK // 16):
                sidx[pl.ds(t * 16, 16)] = idx2[pl.ds(t * 16, 16)]

            @plsc.parallel_loop(0, CHUNK, unroll=4)
            def _(e):
                c = s1v[e] + s2v[e]
                cl = jnp.where(c > 0, c, ALPHA * c)
                w = jnp.exp(-cl)
                w = jnp.where(headmask, w, 0.0)
                wrow[e] = w
                if nheads == 1:
                    wj = lax.gather(
                        w, jnp.zeros((16, 1), jnp.int32),
                        _SPLAT_DNUMS, (1,),
                        mode=lax.GatherScatterMode.PROMISE_IN_BOUNDS)
                    for j in range(NHEADS):
                        obuf[e, pl.ds(j * 16, 16)] = (
                            wj * hbuf[e, pl.ds(j * 16, 16)])
                else:
                    for j in range(NHEADS):
                        wj = lax.gather(
                            w, jnp.full((16, 1), widx[j], jnp.int32),
                            _SPLAT_DNUMS, (1,),
                            mode=lax.GatherScatterMode.PROMISE_IN_BOUNDS)
                        obuf[e, pl.ds(j * 16, 16)] = (
                            wj * hbuf[e, pl.ds(j * 16, 16)])

        def issue_scatters(b):
            sidx, _, obuf, _, _, wrow, _, sems = bufs[b]
            pltpu.async_copy(obuf, acc_h.at[sidx], sems, add=True)
            pltpu.async_copy(wrow, acc_w.at[sidx], sems, add=True)

        def wait_scatters(b):
            sidx, _, obuf, _, _, wrow, _, sems = bufs[b]
            pltpu.make_async_copy(obuf, acc_h.at[sidx], sems).wait()
            pltpu.make_async_copy(wrow, acc_w.at[sidx], sems).wait()

        for k0 in range(NBUF):
            pltpu.sync_copy(packed_hbm.at[k0 * NW + wid], ibufs[k0][0])
            issue_gathers(k0 % NBUF, k0)
        for k0 in range(NBUF, NIBUF):
            issue_idx(k0, k0)

        def slot_body(g, _):
            for u in range(4):
                k = g * 4 + u
                b = u % NBUF
                i4 = u

                @pl.when(k < nch)
                def _():
                    wait_gathers(b, i4)

                @pl.when(jnp.logical_and(k >= NBUF, k - NBUF < nch))
                def _():
                    wait_scatters(b)

                @pl.when(k < nch)
                def _():
                    compute(b, i4)
                    issue_scatters(b)

                @pl.when(k + NIBUF < nch)
                def _():
                    issue_idx(i4, k + NIBUF)

                @pl.when(k + NBUF < nch)
                def _():
                    wait_idx((u + NBUF) % NIBUF, k + NBUF)
                    issue_gathers(b, (u + NBUF) % NIBUF)
            return 0
        lax.fori_loop(0, _SLOTS_OUTER, slot_body, 0)
        plsc.subcore_barrier()

        for t in range(NRCH // 16 + 1):
            rc = sid + 16 * t

            @pl.when(rc < NRCH)
            def _():
                pltpu.sync_copy(acc_h.at[pl.ds(rc * CHUNK, CHUNK)],
                                outh_hbm.at[cid, pl.ds(rc * CHUNK, CHUNK)])
                pltpu.sync_copy(acc_w.at[pl.ds(rc * CHUNK, CHUNK)],
                                outw_hbm.at[cid, pl.ds(rc * CHUNK, CHUNK)])

        @pl.when(sid == NRCH % 16)
        def _():
            pltpu.sync_copy(acc_h.at[pl.ds(NRCH * CHUNK, NRTAIL)],
                            outh_hbm.at[cid, pl.ds(NRCH * CHUNK, NRTAIL)])
            pltpu.sync_copy(acc_w.at[pl.ds(NRCH * CHUNK, NRTAIL)],
                            outw_hbm.at[cid, pl.ds(NRCH * CHUNK, NRTAIL)])

    return edge_pass


_edge_pass8 = _make_edge_pass(NHEADS)
_edge_pass1 = _make_edge_pass(1)


def kernel(Corpus_, batch_inputs, entity_embeddings, edge_list, W, a, W_out, a_out):
    src = edge_list[0].astype(jnp.int32)
    dst = edge_list[1].astype(jnp.int32)
    x = entity_embeddings

    wcat = jnp.transpose(W, (1, 0, 2)).reshape(NFEAT, HD)
    aL = a[:, 0, :NHID]
    aR = a[:, 0, NHID:]
    eyeh = jnp.eye(NHEADS, dtype=jnp.float32)
    a1 = jnp.concatenate(
        [(aL[:, :, None] * eyeh[:, None, :]).reshape(HD, NHEADS),
         jnp.zeros((HD, NHEADS), jnp.float32)], axis=1)
    a2 = jnp.concatenate(
        [(aR[:, :, None] * eyeh[:, None, :]).reshape(HD, NHEADS),
         jnp.zeros((HD, NHEADS), jnp.float32)], axis=1)
    srep = jnp.concatenate(
        [jnp.repeat(eyeh, NHID, axis=1), jnp.zeros((NHEADS, HD), jnp.float32)],
        axis=0)
    a1o = jnp.concatenate(
        [a_out[0, :HD][:, None], jnp.zeros((HD, NHID - 1), jnp.float32)], axis=1)
    a2o = jnp.concatenate(
        [a_out[0, HD:][:, None], jnp.zeros((HD, NHID - 1), jnp.float32)], axis=1)
    s0 = jnp.concatenate(
        [jnp.ones((1, HD), jnp.float32), jnp.zeros((NHID - 1, HD), jnp.float32)],
        axis=0)

    packed = jnp.concatenate(
        [src.reshape(NCHUNKS, CHUNK), dst.reshape(NCHUNKS, CHUNK)], axis=1)

    h1, s1t, s2t = _tc_proj(x, wcat, a1, a2)
    ph1, pw1 = _edge_pass8(packed, h1, s1t, s2t)
    h2, s1t2, s2t2 = _tc_mid(ph1, pw1, srep, W_out, a1o, a2o)
    ph2, pw2 = _edge_pass1(packed, h2, s1t2, s2t2)
    return _tc_out(ph2, pw2, s0)


# speedup vs baseline: 1.0648x; 1.0018x over previous
import functools

import jax
import jax.numpy as jnp
from jax import lax
from jax.experimental import pallas as pl
from jax.experimental.pallas import tpu as pltpu
from jax.experimental.pallas import tpu_sc as plsc

N = 10000
E = 320000
NFEAT = 128
NHID = 16
NHEADS = 8
HD = NHEADS * NHID
ALPHA = 0.2
EPS = 1e-16

NW = 32
CHUNK = 64
NCHUNKS = E // CHUNK
NRCH = N // CHUNK
NRTAIL = N - NRCH * CHUNK


def _tc_proj_body(x_ref, wc_ref, a1_ref, a2_ref, h_ref, s1_ref, s2_ref):
    h = jnp.dot(x_ref[...], wc_ref[...], preferred_element_type=jnp.float32)
    h_ref[...] = h
    s1_ref[...] = jnp.dot(h, a1_ref[...], preferred_element_type=jnp.float32)
    s2_ref[...] = jnp.dot(h, a2_ref[...], preferred_element_type=jnp.float32)


def _tc_proj(x, wc, a1, a2):
    return pl.pallas_call(
        _tc_proj_body,
        out_shape=(
            jax.ShapeDtypeStruct((N, HD), jnp.float32),
            jax.ShapeDtypeStruct((N, NHID), jnp.float32),
            jax.ShapeDtypeStruct((N, NHID), jnp.float32),
        ),
    )(x, wc, a1, a2)


def _tc_mid_body(ph_ref, pw_ref, srep_ref, wout_ref, a1_ref, a2_ref,
                 h2_ref, s1_ref, s2_ref):
    hsum = ph_ref[0] + ph_ref[1]
    wsum = pw_ref[0] + pw_ref[1]
    wfull = jnp.dot(wsum, srep_ref[...], preferred_element_type=jnp.float32)
    hp = hsum / (wfull + EPS)
    x1 = jnp.where(hp > 0, hp, jnp.exp(hp) - 1.0)
    h2 = jnp.dot(x1, wout_ref[...], preferred_element_type=jnp.float32)
    h2_ref[...] = h2
    s1_ref[...] = jnp.dot(h2, a1_ref[...], preferred_element_type=jnp.float32)
    s2_ref[...] = jnp.dot(h2, a2_ref[...], preferred_element_type=jnp.float32)


def _tc_mid(ph, pw, srep, wout, a1, a2):
    return pl.pallas_call(
        _tc_mid_body,
        out_shape=(
            jax.ShapeDtypeStruct((N, HD), jnp.float32),
            jax.ShapeDtypeStruct((N, NHID), jnp.float32),
            jax.ShapeDtypeStruct((N, NHID), jnp.float32),
        ),
    )(ph, pw, srep, wout, a1, a2)


def _tc_out_body(ph_ref, pw_ref, s0_ref, out_ref):
    hsum = ph_ref[0] + ph_ref[1]
    wsum = pw_ref[0] + pw_ref[1]
    wfull = jnp.dot(wsum, s0_ref[...], preferred_element_type=jnp.float32)
    hp = hsum / (wfull + EPS)
    out_ref[...] = jnp.where(hp > 0, hp, jnp.exp(hp) - 1.0)


def _tc_out(ph, pw, s0):
    return pl.pallas_call(
        _tc_out_body,
        out_shape=jax.ShapeDtypeStruct((N, HD), jnp.float32),
    )(ph, pw, s0)


_SPLAT_DNUMS = lax.GatherDimensionNumbers(
    offset_dims=(), collapsed_slice_dims=(0,), start_index_map=(0,))

NBUF = 2
NIBUF = 4
_SLOTS_OUTER = (NCHUNKS // NW + 1 + 2 + 3) // 4 + 1


def _make_edge_pass(nheads):
    widx = [j if nheads == NHEADS else 0 for j in range(NHEADS)]
    mesh = plsc.VectorSubcoreMesh(core_axis_name="c", subcore_axis_name="s")

    per_buf = [
        pltpu.VMEM((CHUNK,), jnp.int32),
        pltpu.VMEM((CHUNK, HD), jnp.float32),
        pltpu.VMEM((CHUNK, HD), jnp.float32),
        pltpu.VMEM((CHUNK, NHID), jnp.float32),
        pltpu.VMEM((CHUNK, NHID), jnp.float32),
        pltpu.VMEM((CHUNK, NHID), jnp.float32),
        pltpu.SemaphoreType.DMA,
        pltpu.SemaphoreType.DMA,
    ]
    npb = len(per_buf)
    per_ibuf = [
        pltpu.VMEM((2 * CHUNK,), jnp.int32),
        pltpu.SemaphoreType.DMA,
    ]
    nipb = len(per_ibuf)

    @functools.partial(
        pl.kernel,
        out_type=(
            jax.ShapeDtypeStruct((2, N, HD), jnp.float32),
            jax.ShapeDtypeStruct((2, N, NHID), jnp.float32),
        ),
        mesh=mesh,
        scratch_types=per_buf * NBUF + per_ibuf * NIBUF + [
            pltpu.VMEM_SHARED((N, HD), jnp.float32),
            pltpu.VMEM_SHARED((N, NHID), jnp.float32),
        ],
        compiler_params=pltpu.CompilerParams(use_tc_tiling_on_sc=False),
    )
    def edge_pass(packed_hbm, h_hbm, s1_hbm, s2_hbm, outh_hbm, outw_hbm,
                  *scratch):
        bufs = [scratch[i * npb:(i + 1) * npb] for i in range(NBUF)]
        ioff = npb * NBUF
        ibufs = [scratch[ioff + i * nipb:ioff + (i + 1) * nipb]
                 for i in range(NIBUF)]
        acc_h, acc_w = scratch[ioff + nipb * NIBUF], scratch[ioff + nipb * NIBUF + 1]
        cid = lax.axis_index("c")
        sid = lax.axis_index("s")
        wid = sid * 2 + cid

        lane = lax.iota(jnp.int32, 16)
        headmask = lane < nheads
        zero16 = jnp.zeros((16,), jnp.float32)

        zh, zw = bufs[0][1], bufs[0][5]

        def zbody(i, _):
            for j in range(NHEADS):
                zh[i, pl.ds(j * 16, 16)] = zero16
            zw[i] = zero16
            return 0
        lax.fori_loop(0, CHUNK, zbody, 0)
        for t in range(NRCH // 16 + 1):
            rc = sid + 16 * t

            @pl.when(rc < NRCH)
            def _():
                pltpu.sync_copy(zh, acc_h.at[pl.ds(rc * CHUNK, CHUNK)])
                pltpu.sync_copy(zw, acc_w.at[pl.ds(rc * CHUNK, CHUNK)])

        @pl.when(sid == NRCH % 16)
        def _():
            pltpu.sync_copy(zh.at[pl.ds(0, NRTAIL)],
                            acc_h.at[pl.ds(NRCH * CHUNK, NRTAIL)])
            pltpu.sync_copy(zw.at[pl.ds(0, NRTAIL)],
                            acc_w.at[pl.ds(NRCH * CHUNK, NRTAIL)])
        plsc.subcore_barrier()

        nch = jnp.where(wid < NCHUNKS - (NCHUNKS // NW) * NW,
                        NCHUNKS // NW + 1, NCHUNKS // NW)

        def issue_idx(i4, k):
            idx2, semi = ibufs[i4]
            pltpu.async_copy(packed_hbm.at[k * NW + wid], idx2, semi)

        def wait_idx(i4, k):
            idx2, semi = ibufs[i4]
            pltpu.make_async_copy(packed_hbm.at[k * NW + wid], idx2, semi).wait()

        def issue_gathers(b, i4):
            _, hbuf, _, s1v, s2v, _, semg, _ = bufs[b]
            idx2 = ibufs[i4][0]
            pltpu.async_copy(h_hbm.at[idx2.at[pl.ds(CHUNK, CHUNK)]], hbuf, semg)
            pltpu.async_copy(s1_hbm.at[idx2.at[pl.ds(0, CHUNK)]], s1v, semg)
            pltpu.async_copy(s2_hbm.at[idx2.at[pl.ds(CHUNK, CHUNK)]], s2v, semg)

        def wait_gathers(b, i4):
            _, hbuf, _, s1v, s2v, _, semg, _ = bufs[b]
            idx2 = ibufs[i4][0]
            pltpu.make_async_copy(
                h_hbm.at[idx2.at[pl.ds(CHUNK, CHUNK)]], hbuf, semg).wait()
            pltpu.make_async_copy(
                s1_hbm.at[idx2.at[pl.ds(0, CHUNK)]], s1v, semg).wait()
            pltpu.make_async_copy(
                s2_hbm.at[idx2.at[pl.ds(CHUNK, CHUNK)]], s2v, semg).wait()

        def compute(b, i4):
            sidx, hbuf, obuf, s1v, s2v, wrow, _, _ = bufs[b]
            idx2 = ibufs[i4][0]
            for t in range(CHUNK // 16):
                sidx[pl.ds(t * 16, 16)] = idx2[pl.ds(t * 16, 16)]

            @plsc.parallel_loop(0, CHUNK, unroll=4)
            def _(e):
                c = s1v[e] + s2v[e]
                cl = jnp.where(c > 0, c, ALPHA * c)
                w = jnp.exp(-cl)
                w = jnp.where(headmask, w, 0.0)
                wrow[e] = w
                if nheads == 1:
                    wj = lax.gather(
                        w, jnp.zeros((16, 1), jnp.int32),
                        _SPLAT_DNUMS, (1,),
                        mode=lax.GatherScatterMode.PROMISE_IN_BOUNDS)
                    for j in range(NHEADS):
                        obuf[e, pl.ds(j * 16, 16)] = (
                            wj * hbuf[e, pl.ds(j * 16, 16)])
                else:
                    for j in range(NHEADS):
                        wj = lax.gather(
                            w, jnp.full((16, 1), widx[j], jnp.int32),
                            _SPLAT_DNUMS, (1,),
                            mode=lax.GatherScatterMode.PROMISE_IN_BOUNDS)
                        obuf[e, pl.ds(j * 16, 16)] = (
                            wj * hbuf[e, pl.ds(j * 16, 16)])

        def issue_scatters(b):
            sidx, _, obuf, _, _, wrow, _, sems = bufs[b]
            pltpu.async_copy(obuf, acc_h.at[sidx], sems, add=True)
            pltpu.async_copy(wrow, acc_w.at[sidx], sems, add=True)

        def wait_scatters(b):
            sidx, _, obuf, _, _, wrow, _, sems = bufs[b]
            pltpu.make_async_copy(obuf, acc_h.at[sidx], sems).wait()
            pltpu.make_async_copy(wrow, acc_w.at[sidx], sems).wait()

        for k0 in range(NBUF):
            pltpu.sync_copy(packed_hbm.at[k0 * NW + wid], ibufs[k0][0])
            issue_gathers(k0 % NBUF, k0)
        for k0 in range(NBUF, NIBUF):
            issue_idx(k0, k0)

        def slot_body(g, _):
            for u in range(4):
                k = g * 4 + u
                b = u % NBUF
                i4 = u

                @pl.when(k < nch)
                def _():
                    wait_gathers(b, i4)

                @pl.when(jnp.logical_and(k >= NBUF, k - NBUF < nch))
                def _():
                    wait_scatters(b)

                @pl.when(k < nch)
                def _():
                    compute(b, i4)
                    issue_scatters(b)

                @pl.when(k + NIBUF < nch)
                def _():
                    issue_idx(i4, k + NIBUF)

                @pl.when(k + NBUF < nch)
                def _():
                    wait_idx((u + NBUF) % NIBUF, k + NBUF)
                    issue_gathers(b, (u + NBUF) % NIBUF)
            return 0
        lax.fori_loop(0, _SLOTS_OUTER, slot_body, 0)
        plsc.subcore_barrier()

        for t in range(NRCH // 16 + 1):
            rc = sid + 16 * t

            @pl.when(rc < NRCH)
            def _():
                pltpu.sync_copy(acc_h.at[pl.ds(rc * CHUNK, CHUNK)],
                                outh_hbm.at[cid, pl.ds(rc * CHUNK, CHUNK)])
                pltpu.sync_copy(acc_w.at[pl.ds(rc * CHUNK, CHUNK)],
                                outw_hbm.at[cid, pl.ds(rc * CHUNK, CHUNK)])

        @pl.when(sid == NRCH % 16)
        def _():
            pltpu.sync_copy(acc_h.at[pl.ds(NRCH * CHUNK, NRTAIL)],
                            outh_hbm.at[cid, pl.ds(NRCH * CHUNK, NRTAIL)])
            pltpu.sync_copy(acc_w.at[pl.ds(NRCH * CHUNK, NRTAIL)],
                            outw_hbm.at[cid, pl.ds(NRCH * CHUNK, NRTAIL)])

    return edge_pass


_edge_pass8 = _make_edge_pass(NHEADS)
_edge_pass1 = _make_edge_pass(1)


def kernel(Corpus_, batch_inputs, entity_embeddings, edge_list, W, a, W_out, a_out):
    src = edge_list[0].astype(jnp.int32)
    dst = edge_list[1].astype(jnp.int32)
    x = entity_embeddings

    wcat = jnp.transpose(W, (1, 0, 2)).reshape(NFEAT, HD)
    aL = a[:, 0, :NHID]
    aR = a[:, 0, NHID:]
    eyeh = jnp.eye(NHEADS, dtype=jnp.float32)
    a1 = jnp.concatenate(
        [(aL[:, :, None] * eyeh[:, None, :]).reshape(HD, NHEADS),
         jnp.zeros((HD, NHEADS), jnp.float32)], axis=1)
    a2 = jnp.concatenate(
        [(aR[:, :, None] * eyeh[:, None, :]).reshape(HD, NHEADS),
         jnp.zeros((HD, NHEADS), jnp.float32)], axis=1)
    srep = jnp.concatenate(
        [jnp.repeat(eyeh, NHID, axis=1), jnp.zeros((NHEADS, HD), jnp.float32)],
        axis=0)
    a1o = jnp.concatenate(
        [a_out[0, :HD][:, None], jnp.zeros((HD, NHID - 1), jnp.float32)], axis=1)
    a2o = jnp.concatenate(
        [a_out[0, HD:][:, None], jnp.zeros((HD, NHID - 1), jnp.float32)], axis=1)
    s0 = jnp.concatenate(
        [jnp.ones((1, HD), jnp.float32), jnp.zeros((NHID - 1, HD), jnp.float32)],
        axis=0)

    packed = jnp.concatenate(
        [src.reshape(NCHUNKS, CHUNK), dst.reshape(NCHUNKS, CHUNK)], axis=1)

    h1, s1t, s2t = _tc_proj(x, wcat, a1, a2)
    ph1, pw1 = _edge_pass8(packed, h1, s1t, s2t)
    h2, s1t2, s2t2 = _tc_mid(ph1, pw1, srep, W_out, a1o, a2o)
    ph2, pw2 = _edge_pass1(packed, h2, s1t2, s2t2)
    return _tc_out(ph2, pw2, s0)


# speedup vs baseline: 1.0678x; 1.0028x over previous
import functools

import jax
import jax.numpy as jnp
from jax import lax
from jax.experimental import pallas as pl
from jax.experimental.pallas import tpu as pltpu
from jax.experimental.pallas import tpu_sc as plsc

N = 10000
E = 320000
NFEAT = 128
NHID = 16
NHEADS = 8
HD = NHEADS * NHID
ALPHA = 0.2
EPS = 1e-16

NW = 32
CHUNK = 64
NCHUNKS = E // CHUNK
NRCH = N // CHUNK
NRTAIL = N - NRCH * CHUNK


def _tc_proj_body(x_ref, wc_ref, a1_ref, a2_ref, h_ref, s1_ref, s2_ref):
    h = jnp.dot(x_ref[...], wc_ref[...], preferred_element_type=jnp.float32)
    h_ref[...] = h
    s1_ref[...] = jnp.dot(h, a1_ref[...], preferred_element_type=jnp.float32)
    s2_ref[...] = jnp.dot(h, a2_ref[...], preferred_element_type=jnp.float32)


def _tc_proj(x, wc, a1, a2):
    return pl.pallas_call(
        _tc_proj_body,
        out_shape=(
            jax.ShapeDtypeStruct((N, HD), jnp.float32),
            jax.ShapeDtypeStruct((N, NHID), jnp.float32),
            jax.ShapeDtypeStruct((N, NHID), jnp.float32),
        ),
    )(x, wc, a1, a2)


def _tc_mid_body(ph_ref, pw_ref, srep_ref, wout_ref, a1_ref, a2_ref,
                 h2_ref, s1_ref, s2_ref):
    hsum = ph_ref[0] + ph_ref[1]
    wsum = pw_ref[0] + pw_ref[1]
    wfull = jnp.dot(wsum, srep_ref[...], preferred_element_type=jnp.float32)
    hp = hsum / (wfull + EPS)
    x1 = jnp.where(hp > 0, hp, jnp.exp(hp) - 1.0)
    h2 = jnp.dot(x1, wout_ref[...], preferred_element_type=jnp.float32)
    h2_ref[...] = h2
    s1_ref[...] = jnp.dot(h2, a1_ref[...], preferred_element_type=jnp.float32)
    s2_ref[...] = jnp.dot(h2, a2_ref[...], preferred_element_type=jnp.float32)


def _tc_mid(ph, pw, srep, wout, a1, a2):
    return pl.pallas_call(
        _tc_mid_body,
        out_shape=(
            jax.ShapeDtypeStruct((N, HD), jnp.float32),
            jax.ShapeDtypeStruct((N, NHID), jnp.float32),
            jax.ShapeDtypeStruct((N, NHID), jnp.float32),
        ),
    )(ph, pw, srep, wout, a1, a2)


def _tc_out_body(ph_ref, pw_ref, s0_ref, out_ref):
    hsum = ph_ref[0] + ph_ref[1]
    wsum = pw_ref[0] + pw_ref[1]
    wfull = jnp.dot(wsum, s0_ref[...], preferred_element_type=jnp.float32)
    hp = hsum / (wfull + EPS)
    out_ref[...] = jnp.where(hp > 0, hp, jnp.exp(hp) - 1.0)


def _tc_out(ph, pw, s0):
    return pl.pallas_call(
        _tc_out_body,
        out_shape=jax.ShapeDtypeStruct((N, HD), jnp.float32),
    )(ph, pw, s0)


_SPLAT_DNUMS = lax.GatherDimensionNumbers(
    offset_dims=(), collapsed_slice_dims=(0,), start_index_map=(0,))

NBUF = 2
NIBUF = 4
_SLOTS_OUTER = (NCHUNKS // NW + 1 + 2 + 3) // 4 + 1


def _make_edge_pass(nheads):
    widx = [j if nheads == NHEADS else 0 for j in range(NHEADS)]
    mesh = plsc.VectorSubcoreMesh(core_axis_name="c", subcore_axis_name="s")

    per_buf = [
        pltpu.VMEM((CHUNK,), jnp.int32),
        pltpu.VMEM((CHUNK, HD), jnp.float32),
        pltpu.VMEM((CHUNK, HD), jnp.float32),
        pltpu.VMEM((CHUNK, NHID), jnp.float32),
        pltpu.VMEM((CHUNK, NHID), jnp.float32),
        pltpu.VMEM((CHUNK, NHID), jnp.float32),
        pltpu.SemaphoreType.DMA,
        pltpu.SemaphoreType.DMA,
    ]
    npb = len(per_buf)
    per_ibuf = [
        pltpu.VMEM((2 * CHUNK,), jnp.int32),
        pltpu.SemaphoreType.DMA,
    ]
    nipb = len(per_ibuf)

    @functools.partial(
        pl.kernel,
        out_type=(
            jax.ShapeDtypeStruct((2, N, HD), jnp.float32),
            jax.ShapeDtypeStruct((2, N, NHID), jnp.float32),
        ),
        mesh=mesh,
        scratch_types=per_buf * NBUF + per_ibuf * NIBUF + [
            pltpu.VMEM_SHARED((N, HD), jnp.float32),
            pltpu.VMEM_SHARED((N, NHID), jnp.float32),
        ],
        compiler_params=pltpu.CompilerParams(use_tc_tiling_on_sc=False),
    )
    def edge_pass(packed_hbm, h_hbm, s1_hbm, s2_hbm, outh_hbm, outw_hbm,
                  *scratch):
        bufs = [scratch[i * npb:(i + 1) * npb] for i in range(NBUF)]
        ioff = npb * NBUF
        ibufs = [scratch[ioff + i * nipb:ioff + (i + 1) * nipb]
                 for i in range(NIBUF)]
        acc_h, acc_w = scratch[ioff + nipb * NIBUF], scratch[ioff + nipb * NIBUF + 1]
        cid = lax.axis_index("c")
        sid = lax.axis_index("s")
        wid = sid * 2 + cid

        lane = lax.iota(jnp.int32, 16)
        headmask = lane < nheads
        zero16 = jnp.zeros((16,), jnp.float32)


        nch = jnp.where(wid < NCHUNKS - (NCHUNKS // NW) * NW,
                        NCHUNKS // NW + 1, NCHUNKS // NW)

        def issue_idx(i4, k):
            idx2, semi = ibufs[i4]
            pltpu.async_copy(packed_hbm.at[k * NW + wid], idx2, semi)

        def wait_idx(i4, k):
            idx2, semi = ibufs[i4]
            pltpu.make_async_copy(packed_hbm.at[k * NW + wid], idx2, semi).wait()

        def issue_gathers(b, i4):
            _, hbuf, _, s1v, s2v, _, semg, _ = bufs[b]
            idx2 = ibufs[i4][0]
            pltpu.async_copy(h_hbm.at[idx2.at[pl.ds(CHUNK, CHUNK)]], hbuf, semg)
            pltpu.async_copy(s1_hbm.at[idx2.at[pl.ds(0, CHUNK)]], s1v, semg)
            pltpu.async_copy(s2_hbm.at[idx2.at[pl.ds(CHUNK, CHUNK)]], s2v, semg)

        def wait_gathers(b, i4):
            _, hbuf, _, s1v, s2v, _, semg, _ = bufs[b]
            idx2 = ibufs[i4][0]
            pltpu.make_async_copy(
                h_hbm.at[idx2.at[pl.ds(CHUNK, CHUNK)]], hbuf, semg).wait()
            pltpu.make_async_copy(
                s1_hbm.at[idx2.at[pl.ds(0, CHUNK)]], s1v, semg).wait()
            pltpu.make_async_copy(
                s2_hbm.at[idx2.at[pl.ds(CHUNK, CHUNK)]], s2v, semg).wait()

        def compute(b, i4):
            sidx, hbuf, obuf, s1v, s2v, wrow, _, _ = bufs[b]
            idx2 = ibufs[i4][0]
            for t in range(CHUNK // 16):
                sidx[pl.ds(t * 16, 16)] = idx2[pl.ds(t * 16, 16)]

            @plsc.parallel_loop(0, CHUNK, unroll=4)
            def _(e):
                c = s1v[e] + s2v[e]
                cl = jnp.where(c > 0, c, ALPHA * c)
                w = jnp.exp(-cl)
                w = jnp.where(headmask, w, 0.0)
                wrow[e] = w
                if nheads == 1:
                    wj = lax.gather(
                        w, jnp.zeros((16, 1), jnp.int32),
                        _SPLAT_DNUMS, (1,),
                        mode=lax.GatherScatterMode.PROMISE_IN_BOUNDS)
                    for j in range(NHEADS):
                        obuf[e, pl.ds(j * 16, 16)] = (
                            wj * hbuf[e, pl.ds(j * 16, 16)])
                else:
                    for j in range(NHEADS):
                        wj = lax.gather(
                            w, jnp.full((16, 1), widx[j], jnp.int32),
                            _SPLAT_DNUMS, (1,),
                            mode=lax.GatherScatterMode.PROMISE_IN_BOUNDS)
                        obuf[e, pl.ds(j * 16, 16)] = (
                            wj * hbuf[e, pl.ds(j * 16, 16)])

        def issue_scatters(b):
            sidx, _, obuf, _, _, wrow, _, sems = bufs[b]
            pltpu.async_copy(obuf, acc_h.at[sidx], sems, add=True)
            pltpu.async_copy(wrow, acc_w.at[sidx], sems, add=True)

        def wait_scatters(b):
            sidx, _, obuf, _, _, wrow, _, sems = bufs[b]
            pltpu.make_async_copy(obuf, acc_h.at[sidx], sems).wait()
            pltpu.make_async_copy(wrow, acc_w.at[sidx], sems).wait()

        for k0 in range(NBUF):
            pltpu.sync_copy(packed_hbm.at[k0 * NW + wid], ibufs[k0][0])
            issue_gathers(k0 % NBUF, k0)
        for k0 in range(NBUF, NIBUF):
            issue_idx(k0, k0)

        zh, zw = bufs[0][2], bufs[0][5]

        def zbody(i, _):
            for j in range(NHEADS):
                zh[i, pl.ds(j * 16, 16)] = zero16
            zw[i] = zero16
            return 0
        lax.fori_loop(0, CHUNK, zbody, 0)
        for t in range(NRCH // 16 + 1):
            rc = sid + 16 * t

            @pl.when(rc < NRCH)
            def _():
                pltpu.sync_copy(zh, acc_h.at[pl.ds(rc * CHUNK, CHUNK)])
                pltpu.sync_copy(zw, acc_w.at[pl.ds(rc * CHUNK, CHUNK)])

        @pl.when(sid == NRCH % 16)
        def _():
            pltpu.sync_copy(zh.at[pl.ds(0, NRTAIL)],
                            acc_h.at[pl.ds(NRCH * CHUNK, NRTAIL)])
            pltpu.sync_copy(zw.at[pl.ds(0, NRTAIL)],
                            acc_w.at[pl.ds(NRCH * CHUNK, NRTAIL)])
        plsc.subcore_barrier()

        def slot_body(g, _):
            for u in range(4):
                k = g * 4 + u
                b = u % NBUF
                i4 = u

                @pl.when(k < nch)
                def _():
                    wait_gathers(b, i4)

                @pl.when(jnp.logical_and(k >= NBUF, k - NBUF < nch))
                def _():
                    wait_scatters(b)

                @pl.when(k < nch)
                def _():
                    compute(b, i4)
                    issue_scatters(b)

                @pl.when(k + NIBUF < nch)
                def _():
                    issue_idx(i4, k + NIBUF)

                @pl.when(k + NBUF < nch)
                def _():
                    wait_idx((u + NBUF) % NIBUF, k + NBUF)
                    issue_gathers(b, (u + NBUF) % NIBUF)
            return 0
        lax.fori_loop(0, _SLOTS_OUTER, slot_body, 0)
        plsc.subcore_barrier()

        for t in range(NRCH // 16 + 1):
            rc = sid + 16 * t

            @pl.when(rc < NRCH)
            def _():
                pltpu.sync_copy(acc_h.at[pl.ds(rc * CHUNK, CHUNK)],
                                outh_hbm.at[cid, pl.ds(rc * CHUNK, CHUNK)])
                pltpu.sync_copy(acc_w.at[pl.ds(rc * CHUNK, CHUNK)],
                                outw_hbm.at[cid, pl.ds(rc * CHUNK, CHUNK)])

        @pl.when(sid == NRCH % 16)
        def _():
            pltpu.sync_copy(acc_h.at[pl.ds(NRCH * CHUNK, NRTAIL)],
                            outh_hbm.at[cid, pl.ds(NRCH * CHUNK, NRTAIL)])
            pltpu.sync_copy(acc_w.at[pl.ds(NRCH * CHUNK, NRTAIL)],
                            outw_hbm.at[cid, pl.ds(NRCH * CHUNK, NRTAIL)])

    return edge_pass


_edge_pass8 = _make_edge_pass(NHEADS)
_edge_pass1 = _make_edge_pass(1)


def kernel(Corpus_, batch_inputs, entity_embeddings, edge_list, W, a, W_out, a_out):
    src = edge_list[0].astype(jnp.int32)
    dst = edge_list[1].astype(jnp.int32)
    x = entity_embeddings

    wcat = jnp.transpose(W, (1, 0, 2)).reshape(NFEAT, HD)
    aL = a[:, 0, :NHID]
    aR = a[:, 0, NHID:]
    eyeh = jnp.eye(NHEADS, dtype=jnp.float32)
    a1 = jnp.concatenate(
        [(aL[:, :, None] * eyeh[:, None, :]).reshape(HD, NHEADS),
         jnp.zeros((HD, NHEADS), jnp.float32)], axis=1)
    a2 = jnp.concatenate(
        [(aR[:, :, None] * eyeh[:, None, :]).reshape(HD, NHEADS),
         jnp.zeros((HD, NHEADS), jnp.float32)], axis=1)
    srep = jnp.concatenate(
        [jnp.repeat(eyeh, NHID, axis=1), jnp.zeros((NHEADS, HD), jnp.float32)],
        axis=0)
    a1o = jnp.concatenate(
        [a_out[0, :HD][:, None], jnp.zeros((HD, NHID - 1), jnp.float32)], axis=1)
    a2o = jnp.concatenate(
        [a_out[0, HD:][:, None], jnp.zeros((HD, NHID - 1), jnp.float32)], axis=1)
    s0 = jnp.concatenate(
        [jnp.ones((1, HD), jnp.float32), jnp.zeros((NHID - 1, HD), jnp.float32)],
        axis=0)

    packed = jnp.concatenate(
        [src.reshape(NCHUNKS, CHUNK), dst.reshape(NCHUNKS, CHUNK)], axis=1)

    h1, s1t, s2t = _tc_proj(x, wcat, a1, a2)
    ph1, pw1 = _edge_pass8(packed, h1, s1t, s2t)
    h2, s1t2, s2t2 = _tc_mid(ph1, pw1, srep, W_out, a1o, a2o)
    ph2, pw2 = _edge_pass1(packed, h2, s1t2, s2t2)
    return _tc_out(ph2, pw2, s0)
